# baseline (speedup 1.0000x reference)
import functools

import jax
import jax.numpy as jnp
from jax import lax
from jax.experimental import pallas as pl
from jax.experimental.pallas import tpu as pltpu
from jax.experimental.pallas import tpu_sc as plsc

NUM_FIELDS = 26
FIELD_SIZE = 40000
EMB_DIM = 16
BATCH = 16384
TOTAL = BATCH * NUM_FIELDS
TOTAL_VOCAB = NUM_FIELDS * FIELD_SIZE

NUM_CORES = 2
NUM_SUBCORES = 16
NUM_WORKERS = NUM_CORES * NUM_SUBCORES
PER_WORKER = TOTAL // NUM_WORKERS

GATHER_W = 128
CHUNK_ROWS = 13
CHUNK = GATHER_W * CHUNK_ROWS
BTILE = 128
CHUNKS_PER_TILE = BTILE * NUM_FIELDS // CHUNK
TILES_PER_WORKER = PER_WORKER // (BTILE * NUM_FIELDS)
TBUF = NUM_FIELDS * EMB_DIM * BTILE


TGROUP = 13 * 128
NUM_TGROUPS = TOTAL_VOCAB // TGROUP
TG_PER_WORKER = -(-NUM_TGROUPS // NUM_WORKERS)


def _table_transpose(table_t):
    mesh = plsc.VectorSubcoreMesh(core_axis_name="c", subcore_axis_name="s")

    @functools.partial(
        pl.kernel,
        out_type=jax.ShapeDtypeStruct((TOTAL_VOCAB * EMB_DIM,), jnp.float32),
        mesh=mesh,
        scratch_types=[
            pltpu.VMEM((EMB_DIM, TGROUP), jnp.float32),
            pltpu.VMEM((EMB_DIM, TGROUP), jnp.float32),
            pltpu.VMEM((EMB_DIM * TGROUP,), jnp.float32),
            pltpu.VMEM((EMB_DIM * TGROUP,), jnp.float32),
            pltpu.SemaphoreType.DMA,
            pltpu.SemaphoreType.DMA,
        ],
        compiler_params=pltpu.CompilerParams(
            use_tc_tiling_on_sc=True, needs_layout_passes=False
        ),
    )
    def k(tab_hbm, out_hbm, in_v0, in_v1, o_v0, o_v1, sem, osem):
        wid = lax.axis_index("s") * NUM_CORES + lax.axis_index("c")
        lane = lax.iota(jnp.int32, 16)
        ins = (in_v0, in_v1)
        outs = (o_v0, o_v1)
        gsz = EMB_DIM * TGROUP

        def start_in(g, buf):
            pltpu.async_copy(tab_hbm.at[:, pl.ds(g * TGROUP, TGROUP)], buf, sem)

        start_in(wid, in_v0)
        start_in(wid + NUM_WORKERS, in_v1)

        @pl.loop(0, TG_PER_WORKER // 2)
        def _(s):
            for p in range(2):
                i = 2 * s + p
                g = wid + i * NUM_WORKERS

                @pl.when(g < NUM_TGROUPS)
                def _(p=p, i=i, g=g):
                    @pl.when(i >= 2)
                    def _():
                        pltpu.make_async_copy(
                            out_hbm.at[pl.ds(0, gsz)], outs[p], osem
                        ).wait()

                    pltpu.make_async_copy(
                        tab_hbm.at[:, pl.ds(0, TGROUP)], ins[p], sem
                    ).wait()

                    @pl.loop(0, TGROUP, step=16)
                    def _(lb):
                        base = (lb + lane) * EMB_DIM
                        for kk in range(EMB_DIM):
                            plsc.store_scatter(
                                outs[p], [base + kk],
                                ins[p][kk, pl.ds(lb, 16)],
                            )

                    @pl.when(g + 2 * NUM_WORKERS < NUM_TGROUPS)
                    def _():
                        start_in(g + 2 * NUM_WORKERS, ins[p])

                    pltpu.async_copy(
                        outs[p], out_hbm.at[pl.ds(g * gsz, gsz)], osem
                    )

        pltpu.make_async_copy(out_hbm.at[pl.ds(0, gsz)], o_v0, osem).wait()
        pltpu.make_async_copy(out_hbm.at[pl.ds(0, gsz)], o_v1, osem).wait()

    return k(table_t)


def _emb_gather(x_flat, offs, table):
    mesh = plsc.VectorSubcoreMesh(core_axis_name="c", subcore_axis_name="s")

    @functools.partial(
        pl.kernel,
        out_type=jax.ShapeDtypeStruct(
            (NUM_FIELDS * 2, BATCH // BTILE, 8 * BTILE), jnp.float32
        ),
        mesh=mesh,
        scratch_types=[
            pltpu.VMEM((CHUNK,), jnp.int32),
            pltpu.VMEM((CHUNK,), jnp.int32),
            pltpu.VMEM((CHUNK,), jnp.int32),
            pltpu.VMEM((CHUNK, EMB_DIM), jnp.float32),
            pltpu.VMEM((CHUNK, EMB_DIM), jnp.float32),
            pltpu.VMEM((NUM_FIELDS * 2, 8 * BTILE), jnp.float32),
            pltpu.SemaphoreType.DMA,
            pltpu.SemaphoreType.DMA,
        ],
        compiler_params=pltpu.CompilerParams(
            use_tc_tiling_on_sc=False, needs_layout_passes=False
        ),
    )
    def k(x_hbm, off_hbm, table_hbm, out_hbm, idx_v0, idx_v1, off_v,
          rows_v0, rows_v1, t_v, sem, osem):
        wid = lax.axis_index("s") * NUM_CORES + lax.axis_index("c")
        base = wid * PER_WORKER
        pltpu.sync_copy(off_hbm, off_v)
        lane = lax.iota(jnp.int32, 16)
        gvec = lane // 8
        slvec = lane % 8 * BTILE
        jgs = [gvec + 2 * j for j in range(NUM_FIELDS)]
        idxs = (idx_v0, idx_v1)
        rows = (rows_v0, rows_v1)

        def launch_chunk(start, p):
            pltpu.sync_copy(x_hbm.at[pl.ds(start, CHUNK)], idxs[p])

            @pl.loop(0, CHUNK, step=16)
            def _(g):
                sl = pl.ds(g, 16)
                idxs[p][sl] = idxs[p][sl] + off_v[sl]

            for r in range(CHUNK_ROWS):
                pltpu.async_copy(
                    table_hbm.at[idxs[p].at[pl.ds(r * GATHER_W, GATHER_W)]],
                    rows[p].at[pl.ds(r * GATHER_W, GATHER_W)],
                    sem,
                )

        def drain_rows(p):
            pltpu.make_async_copy(
                table_hbm.at[pl.ds(0, CHUNK)], rows[p], sem
            ).wait()

        def scatter_rows(p, l0):
            @pl.loop(0, CHUNK // NUM_FIELDS)
            def _(bi):
                sll = slvec + (l0 + bi)
                row0 = bi * NUM_FIELDS
                for j in range(NUM_FIELDS):
                    plsc.store_scatter(
                        t_v, [jgs[j], sll], rows[p][row0 + j, :]
                    )

        launch_chunk(base, 0)

        @pl.loop(0, TILES_PER_WORKER)
        def _(ct):
            tile_start = base + ct * (BTILE * NUM_FIELDS)
            launch_chunk(tile_start + CHUNK, 1)
            drain_rows(0)

            @pl.when(ct > 0)
            def _():
                pltpu.make_async_copy(out_hbm.at[:, 0], t_v, osem).wait()

            scatter_rows(0, 0)

            @pl.when(ct + 1 < TILES_PER_WORKER)
            def _():
                launch_chunk(tile_start + BTILE * NUM_FIELDS, 0)

            drain_rows(1)
            scatter_rows(1, CHUNK // NUM_FIELDS)
            pltpu.async_copy(t_v, out_hbm.at[:, wid * TILES_PER_WORKER + ct],
                             osem)

        pltpu.make_async_copy(out_hbm.at[:, 0], t_v, osem).wait()

    return k(x_flat, offs, table)


def kernel(x, table):
    x_flat = x.reshape(TOTAL).astype(jnp.int32)
    offs = jnp.arange(CHUNK, dtype=jnp.int32) % NUM_FIELDS * FIELD_SIZE
    table_lin = _table_transpose(table.T)
    out4 = _emb_gather(
        x_flat, offs, table_lin.reshape(TOTAL_VOCAB, EMB_DIM)
    )
    out5 = out4.reshape(NUM_FIELDS, 2, BATCH // BTILE, 8, BTILE)
    return out5.transpose(2, 4, 0, 1, 3).reshape(BATCH, NUM_FIELDS, EMB_DIM)


# speedup vs baseline: 1.0004x; 1.0004x over previous
import functools

import jax
import jax.numpy as jnp
from jax import lax
from jax.experimental import pallas as pl
from jax.experimental.pallas import tpu as pltpu
from jax.experimental.pallas import tpu_sc as plsc

NUM_FIELDS = 26
FIELD_SIZE = 40000
EMB_DIM = 16
BATCH = 16384
TOTAL = BATCH * NUM_FIELDS
TOTAL_VOCAB = NUM_FIELDS * FIELD_SIZE

NUM_CORES = 2
NUM_SUBCORES = 16
NUM_WORKERS = NUM_CORES * NUM_SUBCORES
PER_WORKER = TOTAL // NUM_WORKERS

GATHER_W = 128
CHUNK_ROWS = 13
CHUNK = GATHER_W * CHUNK_ROWS
BTILE = 128
CHUNKS_PER_TILE = BTILE * NUM_FIELDS // CHUNK
TILES_PER_WORKER = PER_WORKER // (BTILE * NUM_FIELDS)


TGROUP = 13 * 128
NUM_TGROUPS = TOTAL_VOCAB // TGROUP
TG_PER_WORKER = -(-NUM_TGROUPS // NUM_WORKERS)


def _table_transpose(table_t):
    mesh = plsc.VectorSubcoreMesh(core_axis_name="c", subcore_axis_name="s")

    @functools.partial(
        pl.kernel,
        out_type=jax.ShapeDtypeStruct((TOTAL_VOCAB * EMB_DIM,), jnp.float32),
        mesh=mesh,
        scratch_types=[
            pltpu.VMEM((EMB_DIM, TGROUP), jnp.float32),
            pltpu.VMEM((EMB_DIM, TGROUP), jnp.float32),
            pltpu.VMEM((EMB_DIM * TGROUP,), jnp.float32),
            pltpu.VMEM((EMB_DIM * TGROUP,), jnp.float32),
            pltpu.SemaphoreType.DMA,
            pltpu.SemaphoreType.DMA,
        ],
        compiler_params=pltpu.CompilerParams(
            use_tc_tiling_on_sc=True, needs_layout_passes=False
        ),
    )
    def k(tab_hbm, out_hbm, in_v0, in_v1, o_v0, o_v1, sem, osem):
        wid = lax.axis_index("s") * NUM_CORES + lax.axis_index("c")
        lane = lax.iota(jnp.int32, 16)
        ins = (in_v0, in_v1)
        outs = (o_v0, o_v1)
        gsz = EMB_DIM * TGROUP

        def start_in(g, buf):
            pltpu.async_copy(tab_hbm.at[:, pl.ds(g * TGROUP, TGROUP)], buf, sem)

        start_in(wid, in_v0)
        start_in(wid + NUM_WORKERS, in_v1)

        @pl.loop(0, TG_PER_WORKER // 2)
        def _(s):
            for p in range(2):
                i = 2 * s + p
                g = wid + i * NUM_WORKERS

                @pl.when(g < NUM_TGROUPS)
                def _(p=p, i=i, g=g):
                    @pl.when(i >= 2)
                    def _():
                        pltpu.make_async_copy(
                            out_hbm.at[pl.ds(0, gsz)], outs[p], osem
                        ).wait()

                    pltpu.make_async_copy(
                        tab_hbm.at[:, pl.ds(0, TGROUP)], ins[p], sem
                    ).wait()

                    @pl.loop(0, TGROUP, step=16)
                    def _(lb):
                        base = (lb + lane) * EMB_DIM
                        for kk in range(EMB_DIM):
                            plsc.store_scatter(
                                outs[p], [base + kk],
                                ins[p][kk, pl.ds(lb, 16)],
                            )

                    @pl.when(g + 2 * NUM_WORKERS < NUM_TGROUPS)
                    def _():
                        start_in(g + 2 * NUM_WORKERS, ins[p])

                    pltpu.async_copy(
                        outs[p], out_hbm.at[pl.ds(g * gsz, gsz)], osem
                    )

        pltpu.make_async_copy(out_hbm.at[pl.ds(0, gsz)], o_v0, osem).wait()
        pltpu.make_async_copy(out_hbm.at[pl.ds(0, gsz)], o_v1, osem).wait()

    return k(table_t)


def _emb_gather(x_flat, offs, table):
    mesh = plsc.VectorSubcoreMesh(core_axis_name="c", subcore_axis_name="s")

    @functools.partial(
        pl.kernel,
        out_type=jax.ShapeDtypeStruct(
            (NUM_FIELDS * 2, BATCH // BTILE, 8 * BTILE), jnp.float32
        ),
        mesh=mesh,
        scratch_types=[
            pltpu.VMEM((CHUNK,), jnp.int32),
            pltpu.VMEM((CHUNK,), jnp.int32),
            pltpu.VMEM((CHUNK,), jnp.int32),
            pltpu.VMEM((CHUNK, EMB_DIM), jnp.float32),
            pltpu.VMEM((CHUNK, EMB_DIM), jnp.float32),
            pltpu.VMEM((NUM_FIELDS * 2, 8 * BTILE), jnp.float32),
            pltpu.SemaphoreType.DMA,
            pltpu.SemaphoreType.DMA,
        ],
        compiler_params=pltpu.CompilerParams(
            use_tc_tiling_on_sc=False, needs_layout_passes=False
        ),
    )
    def k(x_hbm, off_hbm, table_hbm, out_hbm, idx_v0, idx_v1, off_v,
          rows_v0, rows_v1, t_v, sem, osem):
        wid = lax.axis_index("s") * NUM_CORES + lax.axis_index("c")
        base = wid * PER_WORKER
        pltpu.sync_copy(off_hbm, off_v)
        lane = lax.iota(jnp.int32, 16)
        gvec = lane // 8
        slvec = lane % 8 * BTILE
        jgs = [gvec + 2 * j for j in range(NUM_FIELDS)]
        idxs = (idx_v0, idx_v1)
        rows = (rows_v0, rows_v1)

        def launch_chunk(start, p):
            pltpu.sync_copy(x_hbm.at[pl.ds(start, CHUNK)], idxs[p])

            @pl.loop(0, CHUNK, step=16)
            def _(g):
                sl = pl.ds(g, 16)
                idxs[p][sl] = idxs[p][sl] + off_v[sl]

            for r in range(CHUNK_ROWS):
                pltpu.async_copy(
                    table_hbm.at[idxs[p].at[pl.ds(r * GATHER_W, GATHER_W)]],
                    rows[p].at[pl.ds(r * GATHER_W, GATHER_W)],
                    sem,
                )

        def drain_rows(p):
            pltpu.make_async_copy(
                table_hbm.at[pl.ds(0, CHUNK)], rows[p], sem
            ).wait()

        def scatter_rows(p, l0):
            @pl.loop(0, CHUNK // NUM_FIELDS)
            def _(bi):
                sll = slvec + (l0 + bi)
                row0 = bi * NUM_FIELDS
                for j in range(NUM_FIELDS):
                    plsc.store_scatter(
                        t_v, [jgs[j], sll], rows[p][row0 + j, :]
                    )

        launch_chunk(base, 0)

        @pl.loop(0, TILES_PER_WORKER)
        def _(ct):
            tile_start = base + ct * (BTILE * NUM_FIELDS)
            launch_chunk(tile_start + CHUNK, 1)
            drain_rows(0)

            @pl.when(ct > 0)
            def _():
                pltpu.make_async_copy(out_hbm.at[:, 0], t_v, osem).wait()

            scatter_rows(0, 0)

            @pl.when(ct + 1 < TILES_PER_WORKER)
            def _():
                launch_chunk(tile_start + BTILE * NUM_FIELDS, 0)

            drain_rows(1)
            scatter_rows(1, CHUNK // NUM_FIELDS)
            pltpu.async_copy(t_v, out_hbm.at[:, wid * TILES_PER_WORKER + ct],
                             osem)

        pltpu.make_async_copy(out_hbm.at[:, 0], t_v, osem).wait()

    return k(x_flat, offs, table)


def kernel(x, table):
    x_flat = x.reshape(TOTAL).astype(jnp.int32)
    offs = jnp.arange(CHUNK, dtype=jnp.int32) % NUM_FIELDS * FIELD_SIZE
    table_lin = _table_transpose(table.T)
    out4 = _emb_gather(
        x_flat, offs, table_lin.reshape(TOTAL_VOCAB, EMB_DIM)
    )
    out5 = out4.reshape(NUM_FIELDS, 2, BATCH // BTILE, 8, BTILE)
    return out5.transpose(2, 4, 0, 1, 3).reshape(BATCH, NUM_FIELDS, EMB_DIM)
